# baseline (speedup 1.0000x reference)
import functools

import jax
import jax.numpy as jnp
from jax import lax
from jax.experimental import pallas as pl
from jax.experimental.pallas import tpu as pltpu
from jax.experimental.pallas import tpu_sc as plsc

D = 2048
HALF = D // 2
N_TOK = 8192
F_ROWS = 256
S_ROWS = 1025
S_PAD = 1032

NC = 2
NS = 16
NW = NC * NS
ROWS_PER_W = N_TOK // NW
C = 16
CS = 8
NCH = ROWS_PER_W // C
NSLOT = 2
LANES = 16


_HI_MASK = -65536
_BN = 512


def _pack_bf16_words(x):
    bits = lax.bitcast_convert_type(x, jnp.int32) + 0x8000
    parts = []
    for g in range(x.shape[1] // 256):
        seg = bits[:, 256 * g: 256 * (g + 1)]
        lo = lax.shift_right_logical(seg[:, :128], 16)
        hi = seg[:, 128:] & _HI_MASK
        parts.append(hi | lo)
    return jnp.concatenate(parts, axis=1)


def _project_body(fe_ref, se_ref, w_ref, b_ref, fp_ref, sp_ref):
    w = w_ref[...].astype(jnp.bfloat16)
    w1 = w[:, :HALF]
    w2 = w[:, HALF:]
    fe = fe_ref[...].astype(jnp.bfloat16)
    se = se_ref[...].astype(jnp.bfloat16)
    dn = (((1,), (1,)), ((), ()))
    fp = lax.dot_general(fe, w1, dn, preferred_element_type=jnp.float32)
    sp = lax.dot_general(se, w2, dn, preferred_element_type=jnp.float32)
    fp_ref[...] = _pack_bf16_words(fp + b_ref[...])
    sp_ref[...] = _pack_bf16_words(sp)


def _project(frame_emb, spatial_emb, w, b2d):
    grid = (D // _BN,)
    return pl.pallas_call(
        _project_body,
        grid=grid,
        in_specs=[
            pl.BlockSpec((F_ROWS, HALF), lambda i: (0, 0)),
            pl.BlockSpec((S_ROWS, HALF), lambda i: (0, 0)),
            pl.BlockSpec((_BN, D), lambda i: (i, 0)),
            pl.BlockSpec((1, _BN), lambda i: (0, i)),
        ],
        out_specs=[
            pl.BlockSpec((F_ROWS, _BN // 2), lambda i: (0, i)),
            pl.BlockSpec((S_ROWS, _BN // 2), lambda i: (0, i)),
        ],
        out_shape=[
            jax.ShapeDtypeStruct((F_ROWS, D // 2), jnp.int32),
            jax.ShapeDtypeStruct((S_ROWS, D // 2), jnp.int32),
        ],
    )(frame_emb, spatial_emb, w, b2d)


def _gather_add_body(fp_hbm, sp_hbm, fid_hbm, sid_hbm, out_hbm,
                     fid_v, sid_v, *slots_flat):
    bufs = [slots_flat[3 * u: 3 * u + 3] for u in range(NSLOT)]
    sems = [slots_flat[3 * NSLOT + 3 * u: 3 * NSLOT + 3 * u + 3]
            for u in range(NSLOT)]
    wid = lax.axis_index("s") * NC + lax.axis_index("c")
    base = wid * ROWS_PER_W
    c1 = pltpu.async_copy(fid_hbm.at[pl.ds(base, ROWS_PER_W)], fid_v, sems[0][0])
    c2 = pltpu.async_copy(sid_hbm.at[pl.ds(base, ROWS_PER_W)], sid_v, sems[0][1])
    c1.wait()
    c2.wait()

    def issue_gathers(ci, fb, sb, semf, semg):
        off = pl.multiple_of(ci * C, 8)
        pltpu.async_copy(fp_hbm.at[fid_v.at[pl.ds(off, C)]], fb, semf)
        pltpu.async_copy(sp_hbm.at[sid_v.at[pl.ds(off, C)]], sb, semg)

    def wait_gathers(ci, fb, sb, semf, semg):
        off = pl.multiple_of(ci * C, 8)
        pltpu.make_async_copy(fp_hbm.at[fid_v.at[pl.ds(off, C)]], fb, semf).wait()
        pltpu.make_async_copy(sp_hbm.at[sid_v.at[pl.ds(off, C)]], sb, semg).wait()

    def issue_store(ci, h, ob, sem):
        off = pl.multiple_of(ci * C + h * CS, 8)
        pltpu.async_copy(ob, out_hbm.at[pl.ds(base + off, CS)], sem)

    def wait_store(ob, sem):
        pltpu.make_async_copy(ob, out_hbm.at[pl.ds(base, CS)], sem).wait()

    def add_sub(fb, sb, ob, r0):
        def row(r, rc):
            for k in range(D // 32):
                g, kl = divmod(k, 8)
                col = 256 * g + 16 * kl
                fw = fb[r0 + r, pl.ds(k * LANES, LANES)]
                sw = sb[r0 + r, pl.ds(k * LANES, LANES)]
                f_lo = lax.bitcast_convert_type(lax.shift_left(fw, 16), jnp.float32)
                s_lo = lax.bitcast_convert_type(lax.shift_left(sw, 16), jnp.float32)
                f_hi = lax.bitcast_convert_type(fw & _HI_MASK, jnp.float32)
                s_hi = lax.bitcast_convert_type(sw & _HI_MASK, jnp.float32)
                ob[r, pl.ds(col, LANES)] = f_lo + s_lo
                ob[r, pl.ds(col + 128, LANES)] = f_hi + s_hi
            return rc

        lax.fori_loop(0, CS, row, 0, unroll=False)

    issue_gathers(0, bufs[0][0], bufs[0][1], sems[0][0], sems[0][1])
    issue_gathers(1, bufs[1][0], bufs[1][1], sems[1][0], sems[1][1])

    def chunk(ci, carry):
        for u in range(NSLOT):
            fb, sb, _ = bufs[u]
            semf, semg, _ = sems[u]

            @pl.when(ci % NSLOT == u)
            def _():
                wait_gathers(ci, fb, sb, semf, semg)

        for h in range(2):
            ob = bufs[h][2]
            semst = sems[h][2]

            @pl.when(ci >= 1)
            def _():
                wait_store(ob, semst)

            for u in range(NSLOT):
                fb, sb, _ = bufs[u]

                @pl.when(ci % NSLOT == u)
                def _():
                    add_sub(fb, sb, ob, h * CS)

            issue_store(ci, h, ob, semst)

        for u in range(NSLOT):
            fb, sb, _ = bufs[u]
            semf, semg, _ = sems[u]

            @pl.when((ci % NSLOT == u) & (ci + NSLOT < NCH))
            def _():
                issue_gathers(ci + NSLOT, fb, sb, semf, semg)

        return carry

    lax.fori_loop(0, NCH, chunk, 0, unroll=False)
    wait_store(bufs[0][2], sems[0][2])
    wait_store(bufs[1][2], sems[1][2])


@functools.partial(
    pl.kernel,
    out_type=jax.ShapeDtypeStruct((N_TOK, D), jnp.float32),
    mesh=plsc.VectorSubcoreMesh(
        core_axis_name="c", subcore_axis_name="s", num_cores=NC, num_subcores=NS
    ),
    scratch_types=(
        [pltpu.VMEM((ROWS_PER_W,), jnp.int32)] * 2
        + [pltpu.VMEM((C, D // 2), jnp.int32),
           pltpu.VMEM((C, D // 2), jnp.int32),
           pltpu.VMEM((CS, D), jnp.float32)] * NSLOT
        + [pltpu.SemaphoreType.DMA] * (3 * NSLOT)
    ),
)
def _gather_add(fp_hbm, sp_hbm, fid_hbm, sid_hbm, out_hbm,
                fid_v, sid_v, *slots_flat):
    _gather_add_body(fp_hbm, sp_hbm, fid_hbm, sid_hbm, out_hbm,
                     fid_v, sid_v, *slots_flat)


def kernel(frame_ids, spatial_ids, frame_emb, spatial_emb, W, b):
    fid = frame_ids.astype(jnp.int32)
    sid = spatial_ids.astype(jnp.int32)
    b2d = b.reshape(1, D)
    fp, sp = _project(frame_emb, spatial_emb, W, b2d)
    return _gather_add(fp, sp, fid, sid)


# speedup vs baseline: 1.0116x; 1.0116x over previous
import functools

import jax
import jax.numpy as jnp
from jax import lax
from jax.experimental import pallas as pl
from jax.experimental.pallas import tpu as pltpu
from jax.experimental.pallas import tpu_sc as plsc

D = 2048
HALF = D // 2
N_TOK = 8192
F_ROWS = 256
S_ROWS = 1025
S_PAD = 1032

NC = 2
NS = 16
NW = NC * NS
ROWS_PER_W = N_TOK // NW
C = 8
NCH = ROWS_PER_W // C
NSLOT = 3
NTRIP = NCH // NSLOT
LANES = 16


_HI_MASK = -65536
_BN = 1024


def _pack_bf16_words(x):
    bits = lax.bitcast_convert_type(x, jnp.int32) + 0x8000
    parts = []
    for g in range(x.shape[1] // 256):
        seg = bits[:, 256 * g: 256 * (g + 1)]
        lo = lax.shift_right_logical(seg[:, :128], 16)
        hi = seg[:, 128:] & _HI_MASK
        parts.append(hi | lo)
    return jnp.concatenate(parts, axis=1)


def _project_body(fe_ref, se_ref, w_ref, b_ref, fp_ref, sp_ref):
    w = w_ref[...].astype(jnp.bfloat16)
    w1 = w[:, :HALF]
    w2 = w[:, HALF:]
    fe = fe_ref[...].astype(jnp.bfloat16)
    se = se_ref[...].astype(jnp.bfloat16)
    dn = (((1,), (1,)), ((), ()))
    fp = lax.dot_general(fe, w1, dn, preferred_element_type=jnp.float32)
    sp = lax.dot_general(se, w2, dn, preferred_element_type=jnp.float32)
    fp_ref[...] = _pack_bf16_words(fp + b_ref[...])
    sp_ref[...] = _pack_bf16_words(sp)


def _project(frame_emb, spatial_emb, w, b2d):
    grid = (D // _BN,)
    return pl.pallas_call(
        _project_body,
        grid=grid,
        in_specs=[
            pl.BlockSpec((F_ROWS, HALF), lambda i: (0, 0)),
            pl.BlockSpec((S_ROWS, HALF), lambda i: (0, 0)),
            pl.BlockSpec((_BN, D), lambda i: (i, 0)),
            pl.BlockSpec((1, _BN), lambda i: (0, i)),
        ],
        out_specs=[
            pl.BlockSpec((F_ROWS, _BN // 2), lambda i: (0, i)),
            pl.BlockSpec((S_ROWS, _BN // 2), lambda i: (0, i)),
        ],
        out_shape=[
            jax.ShapeDtypeStruct((F_ROWS, D // 2), jnp.int32),
            jax.ShapeDtypeStruct((S_ROWS, D // 2), jnp.int32),
        ],
    )(frame_emb, spatial_emb, w, b2d)


def _gather_add_body(fp_hbm, sp_hbm, fid_hbm, sid_hbm, out_hbm,
                     fid_v, sid_v, *slots_flat):
    bufs = [slots_flat[3 * u: 3 * u + 3] for u in range(NSLOT)]
    sems = [slots_flat[3 * NSLOT + 3 * u: 3 * NSLOT + 3 * u + 3]
            for u in range(NSLOT)]
    wid = lax.axis_index("s") * NC + lax.axis_index("c")
    base = wid * ROWS_PER_W
    c1 = pltpu.async_copy(fid_hbm.at[pl.ds(base, ROWS_PER_W)], fid_v, sems[0][0])
    c2 = pltpu.async_copy(sid_hbm.at[pl.ds(base, ROWS_PER_W)], sid_v, sems[0][1])
    c1.wait()
    c2.wait()

    def issue_gathers(ci, fb, sb, semf, sems):
        off = pl.multiple_of(ci * C, 8)
        pltpu.async_copy(fp_hbm.at[fid_v.at[pl.ds(off, C)]], fb, semf)
        pltpu.async_copy(sp_hbm.at[sid_v.at[pl.ds(off, C)]], sb, sems)

    def wait_gathers(ci, fb, sb, semf, sems):
        off = pl.multiple_of(ci * C, 8)
        pltpu.make_async_copy(fp_hbm.at[fid_v.at[pl.ds(off, C)]], fb, semf).wait()
        pltpu.make_async_copy(sp_hbm.at[sid_v.at[pl.ds(off, C)]], sb, sems).wait()

    def issue_store(ci, ob, sem):
        off = pl.multiple_of(ci * C, 8)
        pltpu.async_copy(ob, out_hbm.at[pl.ds(base + off, C)], sem)

    def wait_store(ob, sem):
        pltpu.make_async_copy(ob, out_hbm.at[pl.ds(base, C)], sem).wait()

    def add_chunk(fb, sb, ob):
        def row(r, rc):
            for k in range(D // 32):
                g, kl = divmod(k, 8)
                col = 256 * g + 16 * kl
                fw = fb[r, pl.ds(k * LANES, LANES)]
                sw = sb[r, pl.ds(k * LANES, LANES)]
                f_lo = lax.bitcast_convert_type(lax.shift_left(fw, 16), jnp.float32)
                s_lo = lax.bitcast_convert_type(lax.shift_left(sw, 16), jnp.float32)
                f_hi = lax.bitcast_convert_type(fw & _HI_MASK, jnp.float32)
                s_hi = lax.bitcast_convert_type(sw & _HI_MASK, jnp.float32)
                ob[r, pl.ds(col, LANES)] = f_lo + s_lo
                ob[r, pl.ds(col + 128, LANES)] = f_hi + s_hi
            return rc

        lax.fori_loop(0, C, row, 0, unroll=False)

    for u in range(NSLOT):
        issue_gathers(u, bufs[u][0], bufs[u][1], sems[u][0], sems[u][1])

    def triple(t, carry):
        for u in range(NSLOT):
            fb, sb, ob = bufs[u]
            semf, semg, semst = sems[u]
            ci = NSLOT * t + u
            wait_gathers(ci, fb, sb, semf, semg)

            @pl.when(t >= 1)
            def _():
                wait_store(ob, semst)

            add_chunk(fb, sb, ob)

            @pl.when(ci + NSLOT < NCH)
            def _():
                issue_gathers(ci + NSLOT, fb, sb, semf, semg)

            issue_store(ci, ob, semst)
        return carry

    lax.fori_loop(0, NTRIP, triple, 0, unroll=False)

    for u, ci in enumerate(range(NSLOT * NTRIP, NCH)):
        fb, sb, ob = bufs[u]
        semf, semg, semst = sems[u]
        wait_gathers(ci, fb, sb, semf, semg)
        wait_store(ob, semst)
        add_chunk(fb, sb, ob)
        issue_store(ci, ob, semst)
    for u in range(NSLOT):
        wait_store(bufs[u][2], sems[u][2])


@functools.partial(
    pl.kernel,
    out_type=jax.ShapeDtypeStruct((N_TOK, D), jnp.float32),
    mesh=plsc.VectorSubcoreMesh(
        core_axis_name="c", subcore_axis_name="s", num_cores=NC, num_subcores=NS
    ),
    scratch_types=(
        [pltpu.VMEM((ROWS_PER_W,), jnp.int32)] * 2
        + [pltpu.VMEM((C, D // 2), jnp.int32),
           pltpu.VMEM((C, D // 2), jnp.int32),
           pltpu.VMEM((C, D), jnp.float32)] * NSLOT
        + [pltpu.SemaphoreType.DMA] * (3 * NSLOT)
    ),
)
def _gather_add(fp_hbm, sp_hbm, fid_hbm, sid_hbm, out_hbm,
                fid_v, sid_v, *slots_flat):
    _gather_add_body(fp_hbm, sp_hbm, fid_hbm, sid_hbm, out_hbm,
                     fid_v, sid_v, *slots_flat)


def kernel(frame_ids, spatial_ids, frame_emb, spatial_emb, W, b):
    fid = frame_ids.astype(jnp.int32)
    sid = spatial_ids.astype(jnp.int32)
    b2d = b.reshape(1, D)
    fp, sp = _project(frame_emb, spatial_emb, W, b2d)
    return _gather_add(fp, sp, fid, sid)


# speedup vs baseline: 1.0201x; 1.0084x over previous
import functools

import jax
import jax.numpy as jnp
from jax import lax
from jax.experimental import pallas as pl
from jax.experimental.pallas import tpu as pltpu
from jax.experimental.pallas import tpu_sc as plsc

D = 2048
HALF = D // 2
N_TOK = 8192
F_ROWS = 256
S_ROWS = 1025

NC = 2
NS = 16
NW = NC * NS
ROWS_PER_W = N_TOK // NW
C = 8
NCH = ROWS_PER_W // C
NSLOT = 3
NTRIP = NCH // NSLOT
LANES = 16


_HI_MASK = -65536
_BN = 512


def _pack_bf16_words(x):
    bits = lax.bitcast_convert_type(x, jnp.int32) + 0x8000
    parts = []
    for g in range(x.shape[1] // 256):
        seg = bits[:, 256 * g: 256 * (g + 1)]
        lo = lax.shift_right_logical(seg[:, :128], 16)
        hi = seg[:, 128:] & _HI_MASK
        parts.append(hi | lo)
    return jnp.concatenate(parts, axis=1)


def _project_body(fe_ref, se_ref, w_ref, b_ref, fp_ref, sp_ref):
    w = w_ref[...].astype(jnp.bfloat16)
    w1 = w[:, :HALF]
    w2 = w[:, HALF:]
    fe = fe_ref[...].astype(jnp.bfloat16)
    se = se_ref[...].astype(jnp.bfloat16)
    dn = (((1,), (1,)), ((), ()))
    fp = lax.dot_general(fe, w1, dn, preferred_element_type=jnp.float32)
    sp = lax.dot_general(se, w2, dn, preferred_element_type=jnp.float32)
    fp_ref[...] = _pack_bf16_words(fp + b_ref[...])
    sp_ref[...] = _pack_bf16_words(sp)


def _project(frame_emb, spatial_emb, w, b2d):
    grid = (D // _BN,)
    return pl.pallas_call(
        _project_body,
        grid=grid,
        in_specs=[
            pl.BlockSpec((F_ROWS, HALF), lambda i: (0, 0)),
            pl.BlockSpec((S_ROWS, HALF), lambda i: (0, 0)),
            pl.BlockSpec((_BN, D), lambda i: (i, 0)),
            pl.BlockSpec((1, _BN), lambda i: (0, i)),
        ],
        out_specs=[
            pl.BlockSpec((F_ROWS, _BN // 2), lambda i: (0, i)),
            pl.BlockSpec((S_ROWS, _BN // 2), lambda i: (0, i)),
        ],
        out_shape=[
            jax.ShapeDtypeStruct((F_ROWS, D // 2), jnp.int32),
            jax.ShapeDtypeStruct((S_ROWS, D // 2), jnp.int32),
        ],
    )(frame_emb, spatial_emb, w, b2d)


def _gather_add_body(fp_hbm, sp_hbm, fid_hbm, sid_hbm, out_hbm,
                     fid_v, sid_v, *slots_flat):
    bufs = [slots_flat[3 * u: 3 * u + 3] for u in range(NSLOT)]
    sems = [slots_flat[3 * NSLOT + 3 * u: 3 * NSLOT + 3 * u + 3]
            for u in range(NSLOT)]
    wid = lax.axis_index("s") * NC + lax.axis_index("c")
    base = wid * ROWS_PER_W
    c1 = pltpu.async_copy(fid_hbm.at[pl.ds(base, ROWS_PER_W)], fid_v, sems[0][0])
    c2 = pltpu.async_copy(sid_hbm.at[pl.ds(base, ROWS_PER_W)], sid_v, sems[0][1])
    c1.wait()
    c2.wait()

    def issue_gathers(ci, fb, sb, semf, sems):
        off = pl.multiple_of(ci * C, 8)
        pltpu.async_copy(fp_hbm.at[fid_v.at[pl.ds(off, C)]], fb, semf)
        pltpu.async_copy(sp_hbm.at[sid_v.at[pl.ds(off, C)]], sb, sems)

    def wait_gathers(ci, fb, sb, semf, sems):
        off = pl.multiple_of(ci * C, 8)
        pltpu.make_async_copy(fp_hbm.at[fid_v.at[pl.ds(off, C)]], fb, semf).wait()
        pltpu.make_async_copy(sp_hbm.at[sid_v.at[pl.ds(off, C)]], sb, sems).wait()

    def issue_store(ci, ob, sem):
        off = pl.multiple_of(ci * C, 8)
        pltpu.async_copy(ob, out_hbm.at[pl.ds(base + off, C)], sem)

    def wait_store(ob, sem):
        pltpu.make_async_copy(ob, out_hbm.at[pl.ds(base, C)], sem).wait()

    def add_chunk(fb, sb, ob):
        def row(r, rc):
            for k in range(D // 32):
                g, kl = divmod(k, 8)
                col = 256 * g + 16 * kl
                fw = fb[r, pl.ds(k * LANES, LANES)]
                sw = sb[r, pl.ds(k * LANES, LANES)]
                f_lo = lax.bitcast_convert_type(lax.shift_left(fw, 16), jnp.float32)
                s_lo = lax.bitcast_convert_type(lax.shift_left(sw, 16), jnp.float32)
                f_hi = lax.bitcast_convert_type(fw & _HI_MASK, jnp.float32)
                s_hi = lax.bitcast_convert_type(sw & _HI_MASK, jnp.float32)
                ob[r, pl.ds(col, LANES)] = f_lo + s_lo
                ob[r, pl.ds(col + 128, LANES)] = f_hi + s_hi
            return rc

        lax.fori_loop(0, C, row, 0, unroll=False)

    for u in range(NSLOT):
        issue_gathers(u, bufs[u][0], bufs[u][1], sems[u][0], sems[u][1])

    def triple(t, carry):
        for u in range(NSLOT):
            fb, sb, ob = bufs[u]
            semf, semg, semst = sems[u]
            ci = NSLOT * t + u
            wait_gathers(ci, fb, sb, semf, semg)

            @pl.when(t >= 1)
            def _():
                wait_store(ob, semst)

            add_chunk(fb, sb, ob)

            @pl.when(ci + NSLOT < NCH)
            def _():
                issue_gathers(ci + NSLOT, fb, sb, semf, semg)

            issue_store(ci, ob, semst)
        return carry

    lax.fori_loop(0, NTRIP, triple, 0, unroll=False)

    for u, ci in enumerate(range(NSLOT * NTRIP, NCH)):
        fb, sb, ob = bufs[u]
        semf, semg, semst = sems[u]
        wait_gathers(ci, fb, sb, semf, semg)
        wait_store(ob, semst)
        add_chunk(fb, sb, ob)
        issue_store(ci, ob, semst)
    for u in range(NSLOT):
        wait_store(bufs[u][2], sems[u][2])


@functools.partial(
    pl.kernel,
    out_type=jax.ShapeDtypeStruct((N_TOK, D), jnp.float32),
    mesh=plsc.VectorSubcoreMesh(
        core_axis_name="c", subcore_axis_name="s", num_cores=NC, num_subcores=NS
    ),
    scratch_types=(
        [pltpu.VMEM((ROWS_PER_W,), jnp.int32)] * 2
        + [pltpu.VMEM((C, D // 2), jnp.int32),
           pltpu.VMEM((C, D // 2), jnp.int32),
           pltpu.VMEM((C, D), jnp.float32)] * NSLOT
        + [pltpu.SemaphoreType.DMA] * (3 * NSLOT)
    ),
)
def _gather_add(fp_hbm, sp_hbm, fid_hbm, sid_hbm, out_hbm,
                fid_v, sid_v, *slots_flat):
    _gather_add_body(fp_hbm, sp_hbm, fid_hbm, sid_hbm, out_hbm,
                     fid_v, sid_v, *slots_flat)


def kernel(frame_ids, spatial_ids, frame_emb, spatial_emb, W, b):
    fid = frame_ids.astype(jnp.int32)
    sid = spatial_ids.astype(jnp.int32)
    b2d = b.reshape(1, D)
    fp, sp = _project(frame_emb, spatial_emb, W, b2d)
    return _gather_add(fp, sp, fid, sid)
